# baseline (speedup 1.0000x reference)
import functools

import jax
import jax.numpy as jnp
from jax import lax
from jax.experimental import pallas as pl
from jax.experimental.pallas import tpu as pltpu
from jax.experimental.pallas import tpu_sc as plsc

T = 32768
D_IN = 12
K = 1024
D = 64
DP = 16
TB = 1024
NC = 2
NS = 16
NW = NC * NS
BPW = T // NW


def _tc_body(x_ref, wenc_ref, benc_ref, cbt_ref, cb_ref, wd_ref, bd_ref,
             ze_ref, ids_ref, cdec_ref):
    i = pl.program_id(0)

    z_e = jnp.dot(x_ref[...], wenc_ref[...],
                  preferred_element_type=jnp.float32) + benc_ref[...]
    ze_ref[...] = z_e

    cbt = cbt_ref[...]
    z2 = jnp.sum(z_e * z_e, axis=1, keepdims=True)
    zc = jnp.dot(z_e, cbt, preferred_element_type=jnp.float32)
    c2 = jnp.sum(cbt * cbt, axis=0, keepdims=True)
    d2 = (z2 - 2.0 * zc) + c2

    m = jnp.min(d2, axis=1, keepdims=True)
    iota = lax.broadcasted_iota(jnp.int32, (TB, K), 1)
    ids = jnp.min(jnp.where(d2 == m, iota, K), axis=1, keepdims=True)
    ids_ref[...] = ids

    @pl.when(i == 0)
    def _():
        cdec_ref[...] = jnp.dot(cb_ref[...], wd_ref[...],
                                preferred_element_type=jnp.float32) + bd_ref[...]


_tc_call = pl.pallas_call(
    _tc_body,
    grid=(T // TB,),
    in_specs=[
        pl.BlockSpec((TB, D_IN), lambda i: (i, 0)),
        pl.BlockSpec((D_IN, D), lambda i: (0, 0)),
        pl.BlockSpec((1, D), lambda i: (0, 0)),
        pl.BlockSpec((D, K), lambda i: (0, 0)),
        pl.BlockSpec((K, D), lambda i: (0, 0)),
        pl.BlockSpec((D, DP), lambda i: (0, 0)),
        pl.BlockSpec((1, DP), lambda i: (0, 0)),
    ],
    out_specs=[
        pl.BlockSpec((TB, D), lambda i: (i, 0)),
        pl.BlockSpec((TB, 1), lambda i: (i, 0)),
        pl.BlockSpec((K, DP), lambda i: (0, 0)),
    ],
    out_shape=[
        jax.ShapeDtypeStruct((T, D), jnp.float32),
        jax.ShapeDtypeStruct((T, 1), jnp.int32),
        jax.ShapeDtypeStruct((K, DP), jnp.float32),
    ],
)


@functools.partial(
    pl.kernel,
    out_type=(jax.ShapeDtypeStruct((T, D), jnp.float32),
              jax.ShapeDtypeStruct((T, DP), jnp.float32)),
    mesh=plsc.VectorSubcoreMesh(core_axis_name="c", subcore_axis_name="s"),
    scratch_types=[
        pltpu.VMEM((BPW,), jnp.int32),
        pltpu.VMEM((BPW, D), jnp.float32),
        pltpu.VMEM((BPW, DP), jnp.float32),
        pltpu.SemaphoreType.DMA,
    ],
    compiler_params=pltpu.CompilerParams(use_tc_tiling_on_sc=False),
)
def _sc_gather(cb_hbm, cdec_hbm, ids_hbm, zq_hbm, xr_hbm,
               idx_v, rows_v, dec_v, sem):
    wid = lax.axis_index("s") * NC + lax.axis_index("c")
    base = wid * BPW
    pltpu.sync_copy(ids_hbm.at[pl.ds(base, BPW)], idx_v)
    pltpu.async_copy(cb_hbm.at[idx_v], rows_v, sem).wait()
    pltpu.sync_copy(rows_v, zq_hbm.at[pl.ds(base, BPW)])
    pltpu.async_copy(cdec_hbm.at[idx_v], dec_v, sem).wait()
    pltpu.sync_copy(dec_v, xr_hbm.at[pl.ds(base, BPW)])


def kernel(x, W_enc, b_enc, codebook, W_dec, b_dec):
    wd_pad = jnp.zeros((D, DP), jnp.float32).at[:, :D_IN].set(W_dec)
    bd_pad = jnp.zeros((1, DP), jnp.float32).at[0, :D_IN].set(b_dec)
    z_e, ids2d, cdec = _tc_call(
        x, W_enc, b_enc.reshape(1, D), codebook.T, codebook, wd_pad, bd_pad)
    ids = ids2d.reshape(T)
    z_q, xr_pad = _sc_gather(codebook, cdec, ids)
    return (xr_pad[:, :D_IN], z_e, z_q)
